# baseline (speedup 1.0000x reference)
import jax
import jax.numpy as jnp
from jax import lax
from jax.experimental import pallas as pl
from jax.experimental.pallas import tpu as pltpu
from jax.experimental.pallas import tpu_sc as plsc

_NUM_EMB = 1000000
_DIM = 64
_NVALS = 81920

_NC = 2
_NS = 16
_L = 16
_NW = _NC * _NS
_BPW = _NVALS // _NW
_CH = 128
_NCH = _BPW // _CH


def _body(v0, v1, t0, t1, emb_out, rem_out, idx_c, rows_v, sem):
    wid = lax.axis_index("s") * _NC + lax.axis_index("c")
    base = wid * _BPW
    for f, (vals, table) in enumerate(((v0, t0), (v1, t1))):
        def step(j, carry, vals=vals, table=table, f=f):
            off = base + j * _CH
            pltpu.sync_copy(vals.at[pl.ds(off, _CH)], idx_c)
            for k in range(_CH // _L):
                sl = pl.ds(k * _L, _L)
                idx_c[sl] = idx_c[sl] % _NUM_EMB
            pltpu.sync_copy(idx_c, rem_out.at[f, pl.ds(off, _CH)])
            pltpu.async_copy(table.at[idx_c], rows_v, sem).wait()
            pltpu.sync_copy(rows_v, emb_out.at[f, pl.ds(off, _CH)])
            return carry

        lax.fori_loop(0, _NCH, step, 0)


def kernel(values_0, values_1, lengths, table_0, table_1):
    del lengths
    call = pl.kernel(
        _body,
        out_type=(
            jax.ShapeDtypeStruct((2, _NVALS, _DIM), jnp.float32),
            jax.ShapeDtypeStruct((2, _NVALS), jnp.int32),
        ),
        mesh=plsc.VectorSubcoreMesh(core_axis_name="c", subcore_axis_name="s"),
        scratch_types=[
            pltpu.VMEM((_CH,), jnp.int32),
            pltpu.VMEM((_CH, _DIM), jnp.float32),
            pltpu.SemaphoreType.DMA,
        ],
        compiler_params=pltpu.CompilerParams(use_tc_tiling_on_sc=False),
    )
    return call(values_0, values_1, table_0, table_1)


# speedup vs baseline: 1.0375x; 1.0375x over previous
import jax
import jax.numpy as jnp
from jax import lax
from jax.experimental import pallas as pl
from jax.experimental.pallas import tpu as pltpu
from jax.experimental.pallas import tpu_sc as plsc

_NUM_EMB = 1000000
_DIM = 64
_NVALS = 81920

_NC = 2
_NS = 16
_L = 16
_NW = _NC * _NS
_BPW = _NVALS // _NW
_CH = 128
_NCH = _BPW // _CH
_NSLOT = 8
_LEAD = 4


def _body(v0, v1, t0, t1, emb_out, rem_out, idx2, rows, gsems, ssems):
    wid = lax.axis_index("s") * _NC + lax.axis_index("c")
    base = wid * _BPW
    for f, (vals, table) in enumerate(((v0, t0), (v1, t1))):
        pltpu.sync_copy(vals.at[wid], idx2)

        def mod_j(j, carry):
            for k in range(_CH // _L):
                sl = (j, pl.ds(k * _L, _L))
                idx2[sl] = idx2[sl] % _NUM_EMB
            return carry

        lax.fori_loop(0, _NCH, mod_j, 0)

        pltpu.sync_copy(idx2, rem_out.at[f, wid])

        gd = [None] * _NSLOT
        sd = [None] * _NSLOT

        def start_gather(j):
            b = j % _NSLOT
            gd[b] = pltpu.async_copy(table.at[idx2.at[j]], rows.at[b],
                                     gsems.at[b])

        for j in range(_LEAD):
            start_gather(j)
        for j in range(_NCH):
            b = j % _NSLOT
            gd[b].wait()
            sd[b] = pltpu.async_copy(
                rows.at[b], emb_out.at[f, pl.ds(base + j * _CH, _CH)],
                ssems.at[b])
            nj = j + _LEAD
            if nj < _NCH:
                nb = nj % _NSLOT
                if sd[nb] is not None:
                    sd[nb].wait()
                start_gather(nj)
        for j in range(_NCH - _NSLOT, _NCH):
            sd[j % _NSLOT].wait()


def kernel(values_0, values_1, lengths, table_0, table_1):
    del lengths
    call = pl.kernel(
        _body,
        out_type=(
            jax.ShapeDtypeStruct((2, _NVALS, _DIM), jnp.float32),
            jax.ShapeDtypeStruct((2, _NW, _NCH, _CH), jnp.int32),
        ),
        mesh=plsc.VectorSubcoreMesh(core_axis_name="c", subcore_axis_name="s"),
        scratch_types=[
            pltpu.VMEM((_NCH, _CH), jnp.int32),
            pltpu.VMEM((_NSLOT, _CH, _DIM), jnp.float32),
            pltpu.SemaphoreType.DMA((_NSLOT,)),
            pltpu.SemaphoreType.DMA((_NSLOT,)),
        ],
        compiler_params=pltpu.CompilerParams(use_tc_tiling_on_sc=False),
    )
    emb, rem = call(
        values_0.reshape(_NW, _NCH, _CH),
        values_1.reshape(_NW, _NCH, _CH),
        table_0, table_1,
    )
    return emb, rem.reshape(2, _NVALS)
